# Initial kernel scaffold; baseline (speedup 1.0000x reference)
import jax
import jax.numpy as jnp
from jax.experimental import pallas as pl


def kernel(tool_ids, tool_embed_weight):
    raise NotImplementedError("write your pallas kernel here")


# speedup vs baseline: 6.1038x; 6.1038x over previous
import functools

import jax
import jax.numpy as jnp
from jax import lax
from jax.experimental import pallas as pl
from jax.experimental.pallas import tpu as pltpu
from jax.experimental.pallas import tpu_sc as plsc

D_MODEL = 128
NUM_CORES = 2
NUM_SUBCORES = 16
NUM_WORKERS = NUM_CORES * NUM_SUBCORES
UNIT = 128
KBLK = 80


def _sc_gather(ids2d, table, *, n_units):
    units_per_worker = n_units // NUM_WORKERS
    n_blocks = units_per_worker // KBLK
    b_out = n_units * UNIT

    mesh = plsc.VectorSubcoreMesh(
        core_axis_name="c", subcore_axis_name="s"
    )

    @functools.partial(
        pl.kernel,
        mesh=mesh,
        out_type=jax.ShapeDtypeStruct((b_out, D_MODEL), jnp.float32),
        scratch_types=[
            pltpu.VMEM((KBLK, UNIT), jnp.int32),
            pltpu.VMEM((UNIT, D_MODEL), jnp.float32),
            pltpu.SemaphoreType.DMA,
        ],
    )
    def k(ids_hbm, table_hbm, out_hbm, idx_v, rows_v, sem):
        wid = lax.axis_index("s") * NUM_CORES + lax.axis_index("c")
        ubase = wid * units_per_worker

        def blk_body(bi, carry):
            row0 = ubase + bi * KBLK
            pltpu.sync_copy(ids_hbm.at[pl.ds(row0, KBLK)], idx_v)

            def unit_body(j, carry2):
                u = row0 + j
                pltpu.async_copy(table_hbm.at[idx_v.at[j]], rows_v, sem).wait()
                pltpu.sync_copy(rows_v, out_hbm.at[pl.ds(u * UNIT, UNIT)])
                return carry2

            return lax.fori_loop(0, KBLK, unit_body, carry)

        lax.fori_loop(0, n_blocks, blk_body, 0)

    return k(ids2d, table)


def kernel(tool_ids, tool_embed_weight):
    s, t = tool_ids.shape
    v, d = tool_embed_weight.shape
    b = s * t
    ids2d = tool_ids.reshape(b // UNIT, UNIT).astype(jnp.int32)
    out = _sc_gather(ids2d, tool_embed_weight, n_units=b // UNIT)
    return out.reshape(s, t, d)


# speedup vs baseline: 6.6537x; 1.0901x over previous
import functools

import jax
import jax.numpy as jnp
from jax import lax
from jax.experimental import pallas as pl
from jax.experimental.pallas import tpu as pltpu
from jax.experimental.pallas import tpu_sc as plsc

D_MODEL = 128
NUM_CORES = 2
NUM_SUBCORES = 16
NUM_WORKERS = NUM_CORES * NUM_SUBCORES
UNIT = 128
KBLK = 160
NBUF = 4
LA = 2


def _sc_gather(ids2d, table, *, n_units):
    upw = n_units // NUM_WORKERS
    n_blocks = upw // KBLK
    n_bodies = upw // NBUF
    b_out = n_units * UNIT

    mesh = plsc.VectorSubcoreMesh(core_axis_name="c", subcore_axis_name="s")

    @functools.partial(
        pl.kernel,
        mesh=mesh,
        out_type=jax.ShapeDtypeStruct((b_out, D_MODEL), jnp.float32),
        scratch_types=[
            pltpu.VMEM((2, KBLK, UNIT), jnp.int32),
            pltpu.VMEM((NBUF, UNIT, D_MODEL), jnp.float32),
        ]
        + [pltpu.SemaphoreType.DMA] * (2 * NBUF),
    )
    def k(ids_hbm, table_hbm, out_hbm, idx_v, rows_v, *sems):
        gsem = sems[:NBUF]
        osem = sems[NBUF:]
        wid = lax.axis_index("s") * NUM_CORES + lax.axis_index("c")
        ubase = wid * upw

        def load_idx(m):
            p = lax.rem(m, 2)
            pltpu.sync_copy(ids_hbm.at[pl.ds(ubase + m * KBLK, KBLK)],
                            idx_v.at[p])

        def issue_gather(v, b):
            blk = lax.div(v, KBLK)
            p = lax.rem(blk, 2)
            r = lax.rem(v, KBLK)
            pltpu.async_copy(table_hbm.at[idx_v.at[p, r]], rows_v.at[b],
                             gsem[b])

        def wait_write(b):
            pltpu.make_async_copy(rows_v.at[b], out_hbm.at[pl.ds(0, UNIT)],
                                  osem[b]).wait()

        load_idx(0)
        for b in range(LA):
            issue_gather(b, b)

        def body(t, carry):
            @pl.when(jnp.logical_and(lax.rem(t, n_bodies // n_blocks)
                                     == n_bodies // n_blocks - 1,
                                     t != n_bodies - 1))
            def _():
                load_idx(lax.div(NBUF * t + NBUF, KBLK))

            for kslot in range(NBUF):
                u = NBUF * t + kslot
                bn = (kslot + LA) % NBUF
                pltpu.make_async_copy(table_hbm.at[idx_v.at[0, 0]],
                                      rows_v.at[kslot], gsem[kslot]).wait()
                pltpu.async_copy(rows_v.at[kslot],
                                 out_hbm.at[pl.ds((ubase + u) * UNIT, UNIT)],
                                 osem[kslot])
                @pl.when(u >= LA)
                def _():
                    wait_write(bn)

                @pl.when(u + LA < upw)
                def _():
                    issue_gather(u + LA, bn)

            return carry

        lax.fori_loop(0, n_bodies, body, 0)

        for v in range(upw - LA, upw):
            wait_write(v % NBUF)

    return k(ids2d, table)


def kernel(tool_ids, tool_embed_weight):
    s, t = tool_ids.shape
    v, d = tool_embed_weight.shape
    b = s * t
    ids2d = tool_ids.reshape(b // UNIT, UNIT).astype(jnp.int32)
    out = _sc_gather(ids2d, tool_embed_weight, n_units=b // UNIT)
    return out.reshape(s, t, d)


# speedup vs baseline: 19.4906x; 2.9293x over previous
import functools

import jax
import jax.numpy as jnp
from jax import lax
from jax.experimental import pallas as pl
from jax.experimental.pallas import tpu as pltpu
from jax.experimental.pallas import tpu_sc as plsc

D_MODEL = 128
NUM_CORES = 2
NUM_SUBCORES = 16
NUM_WORKERS = NUM_CORES * NUM_SUBCORES
UNIT = 128
KBLK = 160
NBUF = 4
LA = 2


def _sc_gather(ids2d, table, *, n_units):
    upw = n_units // NUM_WORKERS
    n_blocks = upw // KBLK
    n_bodies = upw // NBUF
    b_out = n_units * UNIT
    vp = table.shape[0]
    rows_per_tile = vp // NUM_SUBCORES

    mesh = plsc.VectorSubcoreMesh(core_axis_name="c", subcore_axis_name="s")

    @functools.partial(
        pl.kernel,
        mesh=mesh,
        out_type=jax.ShapeDtypeStruct((b_out, D_MODEL), jnp.float32),
        scratch_types=[
            pltpu.VMEM((2, KBLK, UNIT), jnp.int32),
            pltpu.VMEM((NBUF, UNIT, D_MODEL), jnp.float32),
            pltpu.VMEM_SHARED((vp, D_MODEL), jnp.float32),
        ]
        + [pltpu.SemaphoreType.DMA] * (2 * NBUF),
    )
    def k(ids_hbm, table_hbm, out_hbm, idx_v, rows_v, table_sp, *sems):
        gsem = sems[:NBUF]
        osem = sems[NBUF:]
        wid = lax.axis_index("s") * NUM_CORES + lax.axis_index("c")
        ubase = wid * upw

        sid = lax.axis_index("s")
        stage = rows_v.at[0, pl.ds(0, rows_per_tile)]
        pltpu.sync_copy(table_hbm.at[pl.ds(sid * rows_per_tile,
                                           rows_per_tile)], stage)
        pltpu.sync_copy(stage, table_sp.at[pl.ds(sid * rows_per_tile,
                                                 rows_per_tile)])
        plsc.subcore_barrier()

        def load_idx(m):
            p = lax.rem(m, 2)
            pltpu.sync_copy(ids_hbm.at[pl.ds(ubase + m * KBLK, KBLK)],
                            idx_v.at[p])

        def issue_gather(v, b):
            blk = lax.div(v, KBLK)
            p = lax.rem(blk, 2)
            r = lax.rem(v, KBLK)
            pltpu.async_copy(table_sp.at[idx_v.at[p, r]], rows_v.at[b],
                             gsem[b])

        def wait_write(b):
            pltpu.make_async_copy(rows_v.at[b], out_hbm.at[pl.ds(0, UNIT)],
                                  osem[b]).wait()

        load_idx(0)
        for b in range(LA):
            issue_gather(b, b)

        def body(t, carry):
            @pl.when(jnp.logical_and(lax.rem(t, n_bodies // n_blocks)
                                     == n_bodies // n_blocks - 1,
                                     t != n_bodies - 1))
            def _():
                load_idx(lax.div(NBUF * t + NBUF, KBLK))

            for kslot in range(NBUF):
                u = NBUF * t + kslot
                bn = (kslot + LA) % NBUF
                pltpu.make_async_copy(table_sp.at[idx_v.at[0, 0]],
                                      rows_v.at[kslot], gsem[kslot]).wait()
                pltpu.async_copy(rows_v.at[kslot],
                                 out_hbm.at[pl.ds((ubase + u) * UNIT, UNIT)],
                                 osem[kslot])
                @pl.when(u >= LA)
                def _():
                    wait_write(bn)

                @pl.when(u + LA < upw)
                def _():
                    issue_gather(u + LA, bn)

            return carry

        lax.fori_loop(0, n_bodies, body, 0)

        for v in range(upw - LA, upw):
            wait_write(v % NBUF)

    return k(ids2d, table)


def kernel(tool_ids, tool_embed_weight):
    s, t = tool_ids.shape
    v, d = tool_embed_weight.shape
    b = s * t
    ids2d = tool_ids.reshape(b // UNIT, UNIT).astype(jnp.int32)
    align = 8 * NUM_SUBCORES
    vp = (v + align - 1) // align * align
    table_p = jnp.pad(tool_embed_weight, ((0, vp - v), (0, 0)))
    out = _sc_gather(ids2d, table_p, n_units=b // UNIT)
    return out.reshape(s, t, d)
